# baseline (speedup 1.0000x reference)
import dataclasses
import functools

import jax
import jax.numpy as jnp
from jax import lax
from jax.experimental import pallas as pl
from jax.experimental.pallas import tpu as pltpu
from jax.experimental.pallas import tpu_sc as plsc

N = 10000
D = 128
NC = 2
NS = 16
L = 16
NW = NC * NS
CH = 128
NCHUNK = 80
EPW = NCHUNK * CH
E_PAD = EPW * NW
PAIRS = NCHUNK // 2
N_PAD = 10240
RPS = N // NS
NP_PS = N_PAD // NS

_mesh = plsc.VectorSubcoreMesh(core_axis_name="c", subcore_axis_name="s")

_sc_params = pltpu.CompilerParams()
if "needs_layout_passes" in pltpu.CompilerParams.__dataclass_fields__:
    _sc_params = dataclasses.replace(_sc_params, needs_layout_passes=False)


@functools.partial(
    pl.kernel, mesh=_mesh,
    out_type=jax.ShapeDtypeStruct((NC, N_PAD), jnp.float32),
    compiler_params=_sc_params,
    scratch_types=[
        pltpu.VMEM_SHARED((NS, N_PAD), jnp.float32),
        pltpu.VMEM((N_PAD,), jnp.float32),
        pltpu.VMEM((EPW,), jnp.int32),
        pltpu.VMEM((EPW,), jnp.float32),
        pltpu.VMEM((NS, 640), jnp.float32),
        pltpu.VMEM((640,), jnp.float32),
    ],
)
def _deg_kernel(dst_hbm, ew_hbm, out_hbm, stage, hist, dstf, ewf, redb, outb):
    c = lax.axis_index("c")
    s = lax.axis_index("s")

    @pl.loop(0, N_PAD // L)
    def _(i):
        hist[pl.ds(i * L, L)] = jnp.zeros((L,), jnp.float32)

    base = (c * NS + s) * EPW
    pltpu.sync_copy(dst_hbm.at[pl.ds(base, EPW)], dstf)
    pltpu.sync_copy(ew_hbm.at[pl.ds(base, EPW)], ewf)

    @pl.loop(0, EPW // CH)
    def _(k):
        for g in range(CH // L):
            off = k * CH + g * L
            plsc.addupdate_scatter(hist, [dstf[pl.ds(off, L)]],
                                   ewf[pl.ds(off, L)])

    pltpu.sync_copy(hist, stage.at[s])
    plsc.subcore_barrier()

    pltpu.sync_copy(stage.at[:, pl.ds(s * 640, 640)], redb)

    @pl.loop(0, 640 // L)
    def _(i):
        v = redb[0, pl.ds(i * L, L)]
        for r in range(1, NS):
            v = v + redb[r, pl.ds(i * L, L)]
        outb[pl.ds(i * L, L)] = v

    pltpu.sync_copy(outb, out_hbm.at[c, pl.ds(s * 640, 640)])


@functools.partial(
    pl.kernel, mesh=_mesh,
    out_type=jax.ShapeDtypeStruct((NC, N, D), jnp.float32),
    compiler_params=_sc_params,
    scratch_types=[
        pltpu.VMEM_SHARED((N, D), jnp.float32),
        pltpu.VMEM((CH, D), jnp.float32),
        pltpu.VMEM((CH, D), jnp.float32),
        pltpu.VMEM((EPW,), jnp.int32),
        pltpu.VMEM((1, CH), jnp.int32),
        pltpu.VMEM((1, CH), jnp.int32),
        pltpu.VMEM((1, CH), jnp.float32),
        pltpu.VMEM((1, CH), jnp.float32),
        pltpu.SemaphoreType.DMA,
        pltpu.SemaphoreType.DMA,
        pltpu.SemaphoreType.DMA,
        pltpu.SemaphoreType.DMA,
        pltpu.SemaphoreType.DMA,
        pltpu.SemaphoreType.DMA,
    ],
)
def _agg_kernel(g_hbm, src_hbm, dst_hbm, ew_hbm, out_hbm,
                acc, rowb0, rowb1, srcf, dstb0, dstb1,
                ewb0, ewb1, sg0, sg1, ss0, ss1, si0, si1):
    c = lax.axis_index("c")
    s = lax.axis_index("s")

    @pl.loop(0, CH)
    def _(i):
        for q in range(D // L):
            rowb0[i, pl.ds(q * L, L)] = jnp.zeros((L,), jnp.float32)

    @pl.loop(0, 8)
    def _(t):
        r0 = s * 640 + t * 80

        @pl.when(r0 < N)
        def _():
            pltpu.sync_copy(rowb0.at[pl.ds(0, 80), :],
                            acc.at[pl.ds(r0, 80), :])

    base = (c * NS + s) * EPW
    pltpu.sync_copy(src_hbm.at[pl.ds(base, EPW)], srcf)

    def load_idx(k, dstb, ewb, sem):
        off = base + k * CH
        pltpu.async_copy(dst_hbm.at[pl.ds(off, CH)], dstb.at[0], sem)
        pltpu.async_copy(ew_hbm.at[pl.ds(off, CH)], ewb.at[0], sem)

    def wait_idx(k, dstb, ewb, sem):
        off = base + k * CH
        pltpu.make_async_copy(dst_hbm.at[pl.ds(off, CH)], dstb.at[0],
                              sem).wait()
        pltpu.make_async_copy(ew_hbm.at[pl.ds(off, CH)], ewb.at[0],
                              sem).wait()

    def compute(rowb, ewb):
        @pl.loop(0, CH // L)
        def _(gi):
            nvec = ewb[0, pl.ds(gi * L, L)]
            for lane in range(L):
                nb = lax.broadcast_in_dim(nvec[lane], (L,), ())
                for q in range(D // L):
                    e = gi * L + lane
                    rowb[e, pl.ds(q * L, L)] = rowb[e, pl.ds(q * L, L)] * nb

    plsc.subcore_barrier()

    load_idx(0, dstb0, ewb0, si0)
    wait_idx(0, dstb0, ewb0, si0)
    pltpu.async_copy(g_hbm.at[srcf.at[pl.ds(0, CH)]], rowb0, sg0)

    @pl.loop(0, PAIRS)
    def _(t):
        ka = 2 * t
        kb = 2 * t + 1

        @pl.when(t > 0)
        def _():
            pltpu.make_async_copy(rowb1, acc.at[dstb1.at[0]], ss1).wait()

        pltpu.async_copy(g_hbm.at[srcf.at[pl.ds(kb * CH, CH)]], rowb1, sg1)
        load_idx(kb, dstb1, ewb1, si1)
        pltpu.make_async_copy(g_hbm.at[srcf.at[pl.ds(ka * CH, CH)]],
                              rowb0, sg0).wait()
        compute(rowb0, ewb0)
        pltpu.async_copy(rowb0, acc.at[dstb0.at[0]], ss0, add=True)
        wait_idx(kb, dstb1, ewb1, si1)
        pltpu.make_async_copy(g_hbm.at[srcf.at[pl.ds(kb * CH, CH)]],
                              rowb1, sg1).wait()
        compute(rowb1, ewb1)
        pltpu.make_async_copy(rowb0, acc.at[dstb0.at[0]], ss0).wait()

        @pl.when(t < PAIRS - 1)
        def _():
            load_idx(ka + 2, dstb0, ewb0, si0)
            pltpu.async_copy(g_hbm.at[srcf.at[pl.ds((ka + 2) * CH, CH)]],
                             rowb0, sg0)
            wait_idx(ka + 2, dstb0, ewb0, si0)

        pltpu.async_copy(rowb1, acc.at[dstb1.at[0]], ss1, add=True)

    pltpu.make_async_copy(rowb1, acc.at[dstb1.at[0]], ss1).wait()

    plsc.subcore_barrier()

    @pl.loop(0, 8)
    def _(t):
        r0 = s * 640 + t * 80

        @pl.when(r0 < N)
        def _():
            pltpu.sync_copy(acc.at[pl.ds(r0, 80), :],
                            out_hbm.at[c, pl.ds(r0, 80), :])


def _dinv_kernel(degp):
    def body(d_ref, o_ref):
        dsum = d_ref[0] + d_ref[1] + 1.0
        o_ref[...] = lax.rsqrt(jnp.maximum(dsum, 1e-12))

    return pl.pallas_call(
        body,
        grid=(1,),
        in_specs=[pl.BlockSpec((2, 80, 128), lambda i: (0, 0, 0))],
        out_specs=pl.BlockSpec((80, 128), lambda i: (0, 0)),
        out_shape=jax.ShapeDtypeStruct((80, 128), jnp.float32),
    )(degp)


def _mm1(x, W, dcol):
    def body(x_ref, w_ref, d_ref, o_ref):
        o_ref[...] = d_ref[...] * jnp.dot(x_ref[...], w_ref[...],
                                          preferred_element_type=jnp.float32)

    return pl.pallas_call(
        body,
        grid=(10,),
        in_specs=[pl.BlockSpec((1000, D), lambda i: (i, 0)),
                  pl.BlockSpec((D, D), lambda i: (0, 0)),
                  pl.BlockSpec((1000, 1), lambda i: (i, 0))],
        out_specs=pl.BlockSpec((1000, D), lambda i: (i, 0)),
        out_shape=jax.ShapeDtypeStruct((N, D), jnp.float32),
    )(x, W, dcol)


def _mm_fused(p, b, W, dcol, gs):
    def body(p0_ref, p1_ref, b_ref, w_ref, d_ref, g_ref, o_ref):
        h = jnp.maximum(
            d_ref[...] * (p0_ref[0] + p1_ref[0] + g_ref[...]) + b_ref[...],
            0.0)
        o_ref[...] = d_ref[...] * jnp.dot(h, w_ref[...],
                                          preferred_element_type=jnp.float32)

    return pl.pallas_call(
        body,
        grid=(10,),
        in_specs=[pl.BlockSpec((1, 1000, D), lambda i: (0, i, 0)),
                  pl.BlockSpec((1, 1000, D), lambda i: (1, i, 0)),
                  pl.BlockSpec((1, D), lambda i: (0, 0)),
                  pl.BlockSpec((D, D), lambda i: (0, 0)),
                  pl.BlockSpec((1000, 1), lambda i: (i, 0)),
                  pl.BlockSpec((1000, D), lambda i: (i, 0))],
        out_specs=pl.BlockSpec((1000, D), lambda i: (i, 0)),
        out_shape=jax.ShapeDtypeStruct((N, D), jnp.float32),
    )(p, p, b, W, dcol, gs)


def _final(p, b, dcol, gs):
    def body(p0_ref, p1_ref, b_ref, d_ref, g_ref, o_ref):
        o_ref[...] = (d_ref[...] * (p0_ref[0] + p1_ref[0] + g_ref[...])
                      + b_ref[...])

    return pl.pallas_call(
        body,
        grid=(10,),
        in_specs=[pl.BlockSpec((1, 1000, D), lambda i: (0, i, 0)),
                  pl.BlockSpec((1, 1000, D), lambda i: (1, i, 0)),
                  pl.BlockSpec((1, D), lambda i: (0, 0)),
                  pl.BlockSpec((1000, 1), lambda i: (i, 0)),
                  pl.BlockSpec((1000, D), lambda i: (i, 0))],
        out_specs=pl.BlockSpec((1000, D), lambda i: (i, 0)),
        out_shape=jax.ShapeDtypeStruct((N, D), jnp.float32),
    )(p, p, b, dcol, gs)


def kernel(x, edge_index, edge_weight, W1, b1, W2, b2, W3, b3):
    pad = E_PAD - edge_index.shape[1]

    def prep(a):
        a = jnp.pad(a, (0, pad))
        return (a.reshape(NCHUNK, NW, CH).transpose(1, 0, 2).reshape(-1))

    src = prep(edge_index[0])
    dst = prep(edge_index[1])
    ew = prep(edge_weight)

    degp = _deg_kernel(dst, ew).reshape(NC, 80, 128)
    dinv = _dinv_kernel(degp)
    dcol = dinv.reshape(N_PAD)[:N].reshape(N, 1)
    g = _mm1(x, W1, dcol)
    p = _agg_kernel(g, src, dst, ew)
    g = _mm_fused(p, b1.reshape(1, D), W2, dcol, g)
    p = _agg_kernel(g, src, dst, ew)
    g = _mm_fused(p, b2.reshape(1, D), W3, dcol, g)
    p = _agg_kernel(g, src, dst, ew)
    return _final(p, b3.reshape(1, D), dcol, g)


# speedup vs baseline: 1.1552x; 1.1552x over previous
import dataclasses
import functools

import jax
import jax.numpy as jnp
from jax import lax
from jax.experimental import pallas as pl
from jax.experimental.pallas import tpu as pltpu
from jax.experimental.pallas import tpu_sc as plsc

N = 10000
D = 128
NC = 2
NS = 16
L = 16
NW = NC * NS
CH = 128
NCHUNK = 82
EPW = NCHUNK * CH
E_PAD = EPW * NW
PAIRS = NCHUNK // 2
N_PAD = 10240
RPS = N // NS
NP_PS = N_PAD // NS

_mesh = plsc.VectorSubcoreMesh(core_axis_name="c", subcore_axis_name="s")

_sc_params = pltpu.CompilerParams()
if "needs_layout_passes" in pltpu.CompilerParams.__dataclass_fields__:
    _sc_params = dataclasses.replace(_sc_params, needs_layout_passes=False)


@functools.partial(
    pl.kernel, mesh=_mesh,
    out_type=jax.ShapeDtypeStruct((NC, N_PAD), jnp.float32),
    compiler_params=_sc_params,
    scratch_types=[
        pltpu.VMEM_SHARED((NS, N_PAD), jnp.float32),
        pltpu.VMEM((N_PAD,), jnp.float32),
        pltpu.VMEM((EPW,), jnp.int32),
        pltpu.VMEM((EPW,), jnp.float32),
        pltpu.VMEM((NS, 640), jnp.float32),
        pltpu.VMEM((640,), jnp.float32),
    ],
)
def _deg_kernel(dst_hbm, ew_hbm, out_hbm, stage, hist, dstf, ewf, redb, outb):
    c = lax.axis_index("c")
    s = lax.axis_index("s")

    @pl.loop(0, N_PAD // L)
    def _(i):
        hist[pl.ds(i * L, L)] = jnp.zeros((L,), jnp.float32)

    base = (c * NS + s) * EPW
    pltpu.sync_copy(dst_hbm.at[pl.ds(base, EPW)], dstf)
    pltpu.sync_copy(ew_hbm.at[pl.ds(base, EPW)], ewf)

    @pl.loop(0, EPW // CH)
    def _(k):
        for g in range(CH // L):
            off = k * CH + g * L
            plsc.addupdate_scatter(hist, [dstf[pl.ds(off, L)]],
                                   ewf[pl.ds(off, L)])

    pltpu.sync_copy(hist, stage.at[s])
    plsc.subcore_barrier()

    pltpu.sync_copy(stage.at[:, pl.ds(s * 640, 640)], redb)

    @pl.loop(0, 640 // L)
    def _(i):
        v = redb[0, pl.ds(i * L, L)]
        for r in range(1, NS):
            v = v + redb[r, pl.ds(i * L, L)]
        outb[pl.ds(i * L, L)] = v

    pltpu.sync_copy(outb, out_hbm.at[c, pl.ds(s * 640, 640)])


@functools.partial(
    pl.kernel, mesh=_mesh,
    out_type=jax.ShapeDtypeStruct((NC, N, D), jnp.float32),
    compiler_params=_sc_params,
    scratch_types=[
        pltpu.VMEM_SHARED((N, D), jnp.float32),
        pltpu.VMEM((CH, D), jnp.float32),
        pltpu.VMEM((CH, D), jnp.float32),
        pltpu.VMEM((EPW,), jnp.int32),
        pltpu.VMEM((1, CH), jnp.int32),
        pltpu.VMEM((1, CH), jnp.int32),
        pltpu.VMEM((1, CH), jnp.float32),
        pltpu.VMEM((1, CH), jnp.float32),
        pltpu.SemaphoreType.DMA,
        pltpu.SemaphoreType.DMA,
        pltpu.SemaphoreType.DMA,
        pltpu.SemaphoreType.DMA,
        pltpu.SemaphoreType.DMA,
        pltpu.SemaphoreType.DMA,
    ],
)
def _agg_kernel(g_hbm, src_hbm, dst_hbm, ew_hbm, out_hbm,
                acc, rowb0, rowb1, srcf, dstb0, dstb1,
                ewb0, ewb1, sg0, sg1, ss0, ss1, si0, si1):
    c = lax.axis_index("c")
    s = lax.axis_index("s")

    @pl.loop(0, CH)
    def _(i):
        for q in range(D // L):
            rowb0[i, pl.ds(q * L, L)] = jnp.zeros((L,), jnp.float32)

    @pl.loop(0, 8)
    def _(t):
        r0 = s * 640 + t * 80

        @pl.when(r0 < N)
        def _():
            pltpu.sync_copy(rowb0.at[pl.ds(0, 80), :],
                            acc.at[pl.ds(r0, 80), :])

    base = (c * NS + s) * EPW
    pltpu.sync_copy(src_hbm.at[pl.ds(base, EPW)], srcf)

    def load_idx(k, dstb, ewb, sem):
        off = base + k * CH
        pltpu.async_copy(dst_hbm.at[pl.ds(off, CH)], dstb.at[0], sem)
        pltpu.async_copy(ew_hbm.at[pl.ds(off, CH)], ewb.at[0], sem)

    def wait_idx(k, dstb, ewb, sem):
        off = base + k * CH
        pltpu.make_async_copy(dst_hbm.at[pl.ds(off, CH)], dstb.at[0],
                              sem).wait()
        pltpu.make_async_copy(ew_hbm.at[pl.ds(off, CH)], ewb.at[0],
                              sem).wait()

    def compute(rowb, ewb):
        @pl.loop(0, CH // L)
        def _(gi):
            nvec = ewb[0, pl.ds(gi * L, L)]
            for lane in range(L):
                nb = lax.broadcast_in_dim(nvec[lane], (L,), ())
                for q in range(D // L):
                    e = gi * L + lane
                    rowb[e, pl.ds(q * L, L)] = rowb[e, pl.ds(q * L, L)] * nb

    plsc.subcore_barrier()

    load_idx(0, dstb0, ewb0, si0)
    wait_idx(0, dstb0, ewb0, si0)
    pltpu.async_copy(g_hbm.at[srcf.at[pl.ds(0, CH)]], rowb0, sg0)

    @pl.loop(0, PAIRS)
    def _(t):
        ka = 2 * t
        kb = 2 * t + 1

        @pl.when(t > 0)
        def _():
            pltpu.make_async_copy(rowb1, acc.at[dstb1.at[0]], ss1).wait()

        pltpu.async_copy(g_hbm.at[srcf.at[pl.ds(kb * CH, CH)]], rowb1, sg1)
        load_idx(kb, dstb1, ewb1, si1)
        pltpu.make_async_copy(g_hbm.at[srcf.at[pl.ds(ka * CH, CH)]],
                              rowb0, sg0).wait()
        compute(rowb0, ewb0)
        pltpu.async_copy(rowb0, acc.at[dstb0.at[0]], ss0, add=True)
        wait_idx(kb, dstb1, ewb1, si1)
        pltpu.make_async_copy(g_hbm.at[srcf.at[pl.ds(kb * CH, CH)]],
                              rowb1, sg1).wait()
        compute(rowb1, ewb1)
        pltpu.make_async_copy(rowb0, acc.at[dstb0.at[0]], ss0).wait()

        @pl.when(t < PAIRS - 1)
        def _():
            load_idx(ka + 2, dstb0, ewb0, si0)
            pltpu.async_copy(g_hbm.at[srcf.at[pl.ds((ka + 2) * CH, CH)]],
                             rowb0, sg0)
            wait_idx(ka + 2, dstb0, ewb0, si0)

        pltpu.async_copy(rowb1, acc.at[dstb1.at[0]], ss1, add=True)

    pltpu.make_async_copy(rowb1, acc.at[dstb1.at[0]], ss1).wait()

    plsc.subcore_barrier()

    @pl.loop(0, 8)
    def _(t):
        r0 = s * 640 + t * 80

        @pl.when(r0 < N)
        def _():
            pltpu.sync_copy(acc.at[pl.ds(r0, 80), :],
                            out_hbm.at[c, pl.ds(r0, 80), :])


def _dinv_kernel(degp):
    def body(d_ref, o_ref):
        dsum = d_ref[0] + d_ref[1]
        o_ref[...] = jnp.where(
            dsum > 0, lax.rsqrt(jnp.maximum(dsum, 1e-12)), 0.0)

    return pl.pallas_call(
        body,
        grid=(1,),
        in_specs=[pl.BlockSpec((2, 80, 128), lambda i: (0, 0, 0))],
        out_specs=pl.BlockSpec((80, 128), lambda i: (0, 0)),
        out_shape=jax.ShapeDtypeStruct((80, 128), jnp.float32),
    )(degp)


def _mm1(x, W, dcol):
    def body(x_ref, w_ref, d_ref, o_ref):
        o_ref[...] = d_ref[...] * jnp.dot(x_ref[...], w_ref[...],
                                          preferred_element_type=jnp.float32)

    return pl.pallas_call(
        body,
        grid=(10,),
        in_specs=[pl.BlockSpec((1000, D), lambda i: (i, 0)),
                  pl.BlockSpec((D, D), lambda i: (0, 0)),
                  pl.BlockSpec((1000, 1), lambda i: (i, 0))],
        out_specs=pl.BlockSpec((1000, D), lambda i: (i, 0)),
        out_shape=jax.ShapeDtypeStruct((N, D), jnp.float32),
    )(x, W, dcol)


def _mm_fused(p, b, W, dcol):
    def body(p0_ref, p1_ref, b_ref, w_ref, d_ref, o_ref):
        h = jnp.maximum(d_ref[...] * (p0_ref[0] + p1_ref[0]) + b_ref[...],
                        0.0)
        o_ref[...] = d_ref[...] * jnp.dot(h, w_ref[...],
                                          preferred_element_type=jnp.float32)

    return pl.pallas_call(
        body,
        grid=(10,),
        in_specs=[pl.BlockSpec((1, 1000, D), lambda i: (0, i, 0)),
                  pl.BlockSpec((1, 1000, D), lambda i: (1, i, 0)),
                  pl.BlockSpec((1, D), lambda i: (0, 0)),
                  pl.BlockSpec((D, D), lambda i: (0, 0)),
                  pl.BlockSpec((1000, 1), lambda i: (i, 0))],
        out_specs=pl.BlockSpec((1000, D), lambda i: (i, 0)),
        out_shape=jax.ShapeDtypeStruct((N, D), jnp.float32),
    )(p, p, b, W, dcol)


def _final(p, b, dcol):
    def body(p0_ref, p1_ref, b_ref, d_ref, o_ref):
        o_ref[...] = d_ref[...] * (p0_ref[0] + p1_ref[0]) + b_ref[...]

    return pl.pallas_call(
        body,
        grid=(10,),
        in_specs=[pl.BlockSpec((1, 1000, D), lambda i: (0, i, 0)),
                  pl.BlockSpec((1, 1000, D), lambda i: (1, i, 0)),
                  pl.BlockSpec((1, D), lambda i: (0, 0)),
                  pl.BlockSpec((1000, 1), lambda i: (i, 0))],
        out_specs=pl.BlockSpec((1000, D), lambda i: (i, 0)),
        out_shape=jax.ShapeDtypeStruct((N, D), jnp.float32),
    )(p, p, b, dcol)


def kernel(x, edge_index, edge_weight, W1, b1, W2, b2, W3, b3):
    loop_idx = jnp.arange(N, dtype=edge_index.dtype)
    pad = E_PAD - (edge_index.shape[1] + N)

    def prep(a):
        a = jnp.pad(a, (0, pad))
        return (a.reshape(NCHUNK, NW, CH).transpose(1, 0, 2).reshape(-1))

    src = prep(jnp.concatenate([edge_index[0], loop_idx]))
    dst = prep(jnp.concatenate([edge_index[1], loop_idx]))
    ew = prep(jnp.concatenate([edge_weight,
                               jnp.ones((N,), edge_weight.dtype)]))

    degp = _deg_kernel(dst, ew).reshape(NC, 80, 128)
    dinv = _dinv_kernel(degp)
    dcol = dinv.reshape(N_PAD)[:N].reshape(N, 1)
    g = _mm1(x, W1, dcol)
    p = _agg_kernel(g, src, dst, ew)
    g = _mm_fused(p, b1.reshape(1, D), W2, dcol)
    p = _agg_kernel(g, src, dst, ew)
    g = _mm_fused(p, b2.reshape(1, D), W3, dcol)
    p = _agg_kernel(g, src, dst, ew)
    return _final(p, b3.reshape(1, D), dcol)


# speedup vs baseline: 1.1693x; 1.0122x over previous
import dataclasses
import functools

import jax
import jax.numpy as jnp
from jax import lax
from jax.experimental import pallas as pl
from jax.experimental.pallas import tpu as pltpu
from jax.experimental.pallas import tpu_sc as plsc

N = 10000
D = 128
NC = 2
NS = 16
L = 16
NW = NC * NS
CH = 128
NCHUNK = 82
EPW = NCHUNK * CH
E_PAD = EPW * NW
PAIRS = NCHUNK // 2
N_PAD = 10240
NP_PS = N_PAD // NS

_mesh = plsc.VectorSubcoreMesh(core_axis_name="c", subcore_axis_name="s")

_sc_params = pltpu.CompilerParams()
if "needs_layout_passes" in pltpu.CompilerParams.__dataclass_fields__:
    _sc_params = dataclasses.replace(_sc_params, needs_layout_passes=False)


@functools.partial(
    pl.kernel, mesh=_mesh,
    out_type=jax.ShapeDtypeStruct((NC, N_PAD), jnp.float32),
    compiler_params=_sc_params,
    scratch_types=[
        pltpu.VMEM_SHARED((NS, N_PAD), jnp.float32),
        pltpu.VMEM((N_PAD,), jnp.float32),
        pltpu.VMEM((EPW,), jnp.int32),
        pltpu.VMEM((EPW,), jnp.float32),
        pltpu.VMEM((NS, 640), jnp.float32),
        pltpu.VMEM((640,), jnp.float32),
    ],
)
def _deg_kernel(dst_hbm, ew_hbm, out_hbm, stage, hist, dstf, ewf, redb, outb):
    c = lax.axis_index("c")
    s = lax.axis_index("s")

    @pl.loop(0, N_PAD // L)
    def _(i):
        hist[pl.ds(i * L, L)] = jnp.zeros((L,), jnp.float32)

    base = (c * NS + s) * EPW
    pltpu.sync_copy(dst_hbm.at[pl.ds(base, EPW)], dstf)
    pltpu.sync_copy(ew_hbm.at[pl.ds(base, EPW)], ewf)

    @pl.loop(0, EPW // CH)
    def _(k):
        for g in range(CH // L):
            off = k * CH + g * L
            plsc.addupdate_scatter(hist, [dstf[pl.ds(off, L)]],
                                   ewf[pl.ds(off, L)])

    pltpu.sync_copy(hist, stage.at[s])
    plsc.subcore_barrier()

    pltpu.sync_copy(stage.at[:, pl.ds(s * 640, 640)], redb)

    @pl.loop(0, 640 // L)
    def _(i):
        v = redb[0, pl.ds(i * L, L)]
        for r in range(1, NS):
            v = v + redb[r, pl.ds(i * L, L)]
        outb[pl.ds(i * L, L)] = v

    pltpu.sync_copy(outb, out_hbm.at[c, pl.ds(s * 640, 640)])


@functools.partial(
    pl.kernel, mesh=_mesh,
    out_type=jax.ShapeDtypeStruct((NC, N, D), jnp.float32),
    compiler_params=_sc_params,
    scratch_types=[
        pltpu.VMEM_SHARED((N, D), jnp.float32),
        pltpu.VMEM((CH, D), jnp.float32),
        pltpu.VMEM((CH, D), jnp.float32),
        pltpu.VMEM((EPW,), jnp.int32),
        pltpu.VMEM((1, CH), jnp.int32),
        pltpu.VMEM((1, CH), jnp.int32),
        pltpu.VMEM((1, CH), jnp.float32),
        pltpu.VMEM((1, CH), jnp.float32),
        pltpu.SemaphoreType.DMA,
        pltpu.SemaphoreType.DMA,
        pltpu.SemaphoreType.DMA,
        pltpu.SemaphoreType.DMA,
        pltpu.SemaphoreType.DMA,
        pltpu.SemaphoreType.DMA,
    ],
)
def _agg_kernel(g_hbm, src_hbm, dst_hbm, ew_hbm, out_hbm,
                acc, rowb0, rowb1, srcf, dstb0, dstb1,
                ewb0, ewb1, sg0, sg1, ss0, ss1, si0, si1):
    c = lax.axis_index("c")
    s = lax.axis_index("s")

    @pl.loop(0, CH)
    def _(i):
        for q in range(D // L):
            rowb0[i, pl.ds(q * L, L)] = jnp.zeros((L,), jnp.float32)

    @pl.loop(0, 8)
    def _(t):
        r0 = s * 640 + t * 80

        @pl.when(r0 < N)
        def _():
            pltpu.sync_copy(rowb0.at[pl.ds(0, 80), :],
                            acc.at[pl.ds(r0, 80), :])

    base = (c * NS + s) * EPW
    pltpu.sync_copy(src_hbm.at[pl.ds(base, EPW)], srcf)

    def load_idx(k, dstb, ewb, sem):
        off = base + k * CH
        pltpu.async_copy(dst_hbm.at[pl.ds(off, CH)], dstb.at[0], sem)
        pltpu.async_copy(ew_hbm.at[pl.ds(off, CH)], ewb.at[0], sem)

    def wait_idx(k, dstb, ewb, sem):
        off = base + k * CH
        pltpu.make_async_copy(dst_hbm.at[pl.ds(off, CH)], dstb.at[0],
                              sem).wait()
        pltpu.make_async_copy(ew_hbm.at[pl.ds(off, CH)], ewb.at[0],
                              sem).wait()

    def compute(rowb, ewb):
        @pl.loop(0, CH // L)
        def _(gi):
            nvec = ewb[0, pl.ds(gi * L, L)]
            for lane in range(L):
                nb = lax.broadcast_in_dim(nvec[lane], (L,), ())
                for q in range(D // L):
                    e = gi * L + lane
                    rowb[e, pl.ds(q * L, L)] = rowb[e, pl.ds(q * L, L)] * nb

    plsc.subcore_barrier()

    load_idx(0, dstb0, ewb0, si0)
    wait_idx(0, dstb0, ewb0, si0)
    pltpu.async_copy(g_hbm.at[srcf.at[pl.ds(0, CH)]], rowb0, sg0)

    @pl.loop(0, PAIRS)
    def _(t):
        ka = 2 * t
        kb = 2 * t + 1

        @pl.when(t > 0)
        def _():
            pltpu.make_async_copy(rowb1, acc.at[dstb1.at[0]], ss1).wait()

        pltpu.async_copy(g_hbm.at[srcf.at[pl.ds(kb * CH, CH)]], rowb1, sg1)
        load_idx(kb, dstb1, ewb1, si1)
        pltpu.make_async_copy(g_hbm.at[srcf.at[pl.ds(ka * CH, CH)]],
                              rowb0, sg0).wait()
        compute(rowb0, ewb0)
        pltpu.async_copy(rowb0, acc.at[dstb0.at[0]], ss0, add=True)
        wait_idx(kb, dstb1, ewb1, si1)
        pltpu.make_async_copy(g_hbm.at[srcf.at[pl.ds(kb * CH, CH)]],
                              rowb1, sg1).wait()
        compute(rowb1, ewb1)
        pltpu.async_copy(rowb1, acc.at[dstb1.at[0]], ss1, add=True)
        pltpu.make_async_copy(rowb0, acc.at[dstb0.at[0]], ss0).wait()

        @pl.when(t < PAIRS - 1)
        def _():
            load_idx(ka + 2, dstb0, ewb0, si0)
            pltpu.async_copy(g_hbm.at[srcf.at[pl.ds((ka + 2) * CH, CH)]],
                             rowb0, sg0)
            wait_idx(ka + 2, dstb0, ewb0, si0)

    pltpu.make_async_copy(rowb1, acc.at[dstb1.at[0]], ss1).wait()

    plsc.subcore_barrier()

    @pl.loop(0, 8)
    def _(t):
        r0 = s * 640 + t * 80

        @pl.when(r0 < N)
        def _():
            pltpu.sync_copy(acc.at[pl.ds(r0, 80), :],
                            out_hbm.at[c, pl.ds(r0, 80), :])


def _dinv_kernel(degp):
    def body(d_ref, o_ref):
        dsum = d_ref[0] + d_ref[1]
        o_ref[...] = jnp.where(
            dsum > 0, lax.rsqrt(jnp.maximum(dsum, 1e-12)), 0.0)

    return pl.pallas_call(
        body,
        grid=(1,),
        in_specs=[pl.BlockSpec((2, 80, 128), lambda i: (0, 0, 0))],
        out_specs=pl.BlockSpec((80, 128), lambda i: (0, 0)),
        out_shape=jax.ShapeDtypeStruct((80, 128), jnp.float32),
    )(degp)


def _mm1(x, W, dcol):
    def body(x_ref, w_ref, d_ref, o_ref):
        o_ref[...] = d_ref[...] * jnp.dot(x_ref[...], w_ref[...],
                                          preferred_element_type=jnp.float32)

    return pl.pallas_call(
        body,
        grid=(10,),
        in_specs=[pl.BlockSpec((1000, D), lambda i: (i, 0)),
                  pl.BlockSpec((D, D), lambda i: (0, 0)),
                  pl.BlockSpec((1000, 1), lambda i: (i, 0))],
        out_specs=pl.BlockSpec((1000, D), lambda i: (i, 0)),
        out_shape=jax.ShapeDtypeStruct((N, D), jnp.float32),
    )(x, W, dcol)


def _mm_fused(p, b, W, dcol):
    def body(p0_ref, p1_ref, b_ref, w_ref, d_ref, o_ref):
        h = jnp.maximum(d_ref[...] * (p0_ref[0] + p1_ref[0]) + b_ref[...],
                        0.0)
        o_ref[...] = d_ref[...] * jnp.dot(h, w_ref[...],
                                          preferred_element_type=jnp.float32)

    return pl.pallas_call(
        body,
        grid=(10,),
        in_specs=[pl.BlockSpec((1, 1000, D), lambda i: (0, i, 0)),
                  pl.BlockSpec((1, 1000, D), lambda i: (1, i, 0)),
                  pl.BlockSpec((1, D), lambda i: (0, 0)),
                  pl.BlockSpec((D, D), lambda i: (0, 0)),
                  pl.BlockSpec((1000, 1), lambda i: (i, 0))],
        out_specs=pl.BlockSpec((1000, D), lambda i: (i, 0)),
        out_shape=jax.ShapeDtypeStruct((N, D), jnp.float32),
    )(p, p, b, W, dcol)


def _final(p, b, dcol):
    def body(p0_ref, p1_ref, b_ref, d_ref, o_ref):
        o_ref[...] = d_ref[...] * (p0_ref[0] + p1_ref[0]) + b_ref[...]

    return pl.pallas_call(
        body,
        grid=(10,),
        in_specs=[pl.BlockSpec((1, 1000, D), lambda i: (0, i, 0)),
                  pl.BlockSpec((1, 1000, D), lambda i: (1, i, 0)),
                  pl.BlockSpec((1, D), lambda i: (0, 0)),
                  pl.BlockSpec((1000, 1), lambda i: (i, 0))],
        out_specs=pl.BlockSpec((1000, D), lambda i: (i, 0)),
        out_shape=jax.ShapeDtypeStruct((N, D), jnp.float32),
    )(p, p, b, dcol)


def kernel(x, edge_index, edge_weight, W1, b1, W2, b2, W3, b3):
    loop_idx = jnp.arange(N, dtype=edge_index.dtype)
    pad = E_PAD - (edge_index.shape[1] + N)

    def prep(a):
        a = jnp.pad(a, (0, pad))
        return (a.reshape(NCHUNK, NW, CH).transpose(1, 0, 2).reshape(-1))

    src = prep(jnp.concatenate([edge_index[0], loop_idx]))
    dst = prep(jnp.concatenate([edge_index[1], loop_idx]))
    ew = prep(jnp.concatenate([edge_weight,
                               jnp.ones((N,), edge_weight.dtype)]))

    degp = _deg_kernel(dst, ew).reshape(NC, 80, 128)
    dinv = _dinv_kernel(degp)
    dcol = dinv.reshape(N_PAD)[:N].reshape(N, 1)
    g = _mm1(x, W1, dcol)
    p = _agg_kernel(g, src, dst, ew)
    g = _mm_fused(p, b1.reshape(1, D), W2, dcol)
    p = _agg_kernel(g, src, dst, ew)
    g = _mm_fused(p, b2.reshape(1, D), W3, dcol)
    p = _agg_kernel(g, src, dst, ew)
    return _final(p, b3.reshape(1, D), dcol)
